# baseline (speedup 1.0000x reference)
import functools
import jax
import jax.numpy as jnp
from jax import lax
from jax.experimental import pallas as pl
from jax.experimental.pallas import tpu as pltpu
from jax.experimental.pallas import tpu_sc as plsc

N, P, D = 10000, 8, 256
ROW = P * D
L = 16
T = D // L
NC, NS = 2, 16
NW = NC * NS
CH = 16
NCHUNK = N // CH
BASE_CH = NCHUNK // NW
EXTRA = NCHUNK - BASE_CH * NW


def _worker_range(wid):
    start = wid * BASE_CH + jnp.minimum(wid, EXTRA)
    count = BASE_CH + jnp.where(wid < EXTRA, 1, 0)
    return start, count


def _sc_mesh():
    return plsc.VectorSubcoreMesh(core_axis_name="c", subcore_axis_name="s")


def _pass1_body(h_hbm, w_hbm, out_hbm, wbuf, buf, stage):
    wid = lax.axis_index("s") * NC + lax.axis_index("c")
    pltpu.sync_copy(w_hbm, wbuf)
    wv = [wbuf[pl.ds(t * L, L)] for t in range(T)]
    start, count = _worker_range(wid)

    def chunk_body(g, acc):
        cid = start + g
        pltpu.sync_copy(h_hbm.at[pl.ds(cid * CH, CH), :], buf)

        def row_body(r, acc):
            acc = list(acc)
            for p in range(P):
                a = acc[p]
                for t in range(T):
                    a = a + buf[r, pl.ds(p * D + t * L, L)] * wv[t]
                acc[p] = a
            return tuple(acc)

        return lax.fori_loop(0, CH, row_body, acc)

    acc0 = tuple(jnp.zeros((L,), jnp.float32) for _ in range(P))
    acc = lax.fori_loop(0, count, chunk_body, acc0)
    for p in range(P):
        stage[p, :] = acc[p]
    pltpu.sync_copy(stage, out_hbm.at[wid])


def _pass1(h2, Wf):
    f = pl.kernel(
        _pass1_body,
        out_type=jax.ShapeDtypeStruct((NW, P, L), jnp.float32),
        mesh=_sc_mesh(),
        scratch_types=[
            pltpu.VMEM((D,), jnp.float32),
            pltpu.VMEM((CH, ROW), jnp.float32),
            pltpu.VMEM((P, L), jnp.float32),
        ],
    )
    return f(h2, Wf)


def _beta_body(wpart_ref, beta_ref):
    w = jnp.sum(wpart_ref[...], axis=(0, 2)).reshape(P, 1)
    w = jnp.where(w >= 0, w, 0.01 * w)
    m = jnp.max(w, axis=0, keepdims=True)
    e = jnp.exp(w - m)
    beta = e / jnp.sum(e, axis=0, keepdims=True)
    beta_ref[...] = jnp.broadcast_to(beta, (P, 128))


def _beta(wpart):
    return pl.pallas_call(
        _beta_body,
        out_shape=jax.ShapeDtypeStruct((P, 128), jnp.float32),
    )(wpart)


def _pass2_body(h_hbm, beta_hbm, z_hbm, bbuf, buf, zbuf):
    wid = lax.axis_index("s") * NC + lax.axis_index("c")
    pltpu.sync_copy(beta_hbm, bbuf)
    bv = [bbuf[p, pl.ds(0, L)] for p in range(P)]
    start, count = _worker_range(wid)

    def chunk_body(g, carry):
        cid = start + g
        pltpu.sync_copy(h_hbm.at[pl.ds(cid * CH, CH), :], buf)

        def row_body(r, carry):
            for t in range(T):
                z = bv[0] * buf[r, pl.ds(0 * D + t * L, L)]
                for p in range(1, P):
                    z = z + bv[p] * buf[r, pl.ds(p * D + t * L, L)]
                zbuf[r, pl.ds(t * L, L)] = z
            return carry

        lax.fori_loop(0, CH, row_body, 0)
        pltpu.sync_copy(zbuf, z_hbm.at[pl.ds(cid * CH, CH), :])
        return carry

    lax.fori_loop(0, count, chunk_body, 0)


def _pass2(h2, beta_b):
    f = pl.kernel(
        _pass2_body,
        out_type=jax.ShapeDtypeStruct((N, D), jnp.float32),
        mesh=_sc_mesh(),
        scratch_types=[
            pltpu.VMEM((P, 128), jnp.float32),
            pltpu.VMEM((CH, ROW), jnp.float32),
            pltpu.VMEM((CH, D), jnp.float32),
        ],
    )
    return f(h2, beta_b)


def kernel(h, W):
    h2 = h.reshape(N, ROW)
    Wf = W.reshape(D)
    wpart = _pass1(h2, Wf)
    beta_b = _beta(wpart)
    return _pass2(h2, beta_b)


# speedup vs baseline: 1.2672x; 1.2672x over previous
import functools
import jax
import jax.numpy as jnp
from jax import lax
from jax.experimental import pallas as pl
from jax.experimental.pallas import tpu as pltpu
from jax.experimental.pallas import tpu_sc as plsc

N, P, D = 10000, 8, 256
ROW = P * D
L = 16
T = D // L
NC, NS = 2, 16
NW = NC * NS
CH = 16
NCHUNK = N // CH
BASE_CH = NCHUNK // NW
EXTRA = NCHUNK - BASE_CH * NW
HALF = (BASE_CH - 1) // 2


def _worker_start(wid):
    return wid * BASE_CH + jnp.minimum(wid, EXTRA)


def _sc_mesh():
    return plsc.VectorSubcoreMesh(core_axis_name="c", subcore_axis_name="s")


def _in_start(h_hbm, cid, buf, sem):
    pltpu.async_copy(h_hbm.at[pl.ds(cid * CH, CH), :], buf, sem)


def _in_wait(h_hbm, buf, sem):
    pltpu.make_async_copy(h_hbm.at[pl.ds(0, CH), :], buf, sem).wait()


def _pass1_body(h_hbm, w_hbm, out_hbm, wbuf, buf0, buf1, stage, sem0, sem1):
    wid = lax.axis_index("s") * NC + lax.axis_index("c")
    pltpu.sync_copy(w_hbm, wbuf)
    wv = [wbuf[pl.ds(t * L, L)] for t in range(T)]
    start = _worker_start(wid)

    def rows(buf, acc):
        def row_body(r, acc):
            acc = list(acc)
            for p in range(P):
                a = acc[p]
                for t in range(T):
                    a = a + buf[r, pl.ds(p * D + t * L, L)] * wv[t]
                acc[p] = a
            return tuple(acc)
        return lax.fori_loop(0, CH, row_body, acc)

    _in_start(h_hbm, start, buf0, sem0)

    def body2(k, acc):
        g = 2 * k
        _in_start(h_hbm, start + g + 1, buf1, sem1)
        _in_wait(h_hbm, buf0, sem0)
        acc = rows(buf0, acc)
        _in_start(h_hbm, start + g + 2, buf0, sem0)
        _in_wait(h_hbm, buf1, sem1)
        return rows(buf1, acc)

    acc0 = tuple(jnp.zeros((L,), jnp.float32) for _ in range(P))
    acc = lax.fori_loop(0, HALF, body2, acc0)
    _in_wait(h_hbm, buf0, sem0)
    acc = rows(buf0, acc)
    for p in range(P):
        stage[p, :] = acc[p]

    @pl.when(wid < EXTRA)
    def _extra():
        pltpu.sync_copy(h_hbm.at[pl.ds((start + BASE_CH) * CH, CH), :], buf1)
        acc_e = rows(buf1, acc0)
        for p in range(P):
            stage[p, :] = stage[p, :] + acc_e[p]

    pltpu.sync_copy(stage, out_hbm.at[wid])


def _pass1(h2, Wf):
    f = pl.kernel(
        _pass1_body,
        out_type=jax.ShapeDtypeStruct((NW, P, L), jnp.float32),
        mesh=_sc_mesh(),
        scratch_types=[
            pltpu.VMEM((D,), jnp.float32),
            pltpu.VMEM((CH, ROW), jnp.float32),
            pltpu.VMEM((CH, ROW), jnp.float32),
            pltpu.VMEM((P, L), jnp.float32),
            pltpu.SemaphoreType.DMA,
            pltpu.SemaphoreType.DMA,
        ],
    )
    return f(h2, Wf)


def _beta_body(wpart_ref, beta_ref):
    w = jnp.sum(wpart_ref[...], axis=(0, 2)).reshape(P, 1)
    w = jnp.where(w >= 0, w, 0.01 * w)
    m = jnp.max(w, axis=0, keepdims=True)
    e = jnp.exp(w - m)
    beta = e / jnp.sum(e, axis=0, keepdims=True)
    beta_ref[...] = jnp.broadcast_to(beta, (P, 128))


def _beta(wpart):
    return pl.pallas_call(
        _beta_body,
        out_shape=jax.ShapeDtypeStruct((P, 128), jnp.float32),
    )(wpart)


def _pass2_body(h_hbm, beta_hbm, z_hbm, bbuf, buf0, buf1, zb0, zb1,
                sem0, sem1, osem0, osem1):
    wid = lax.axis_index("s") * NC + lax.axis_index("c")
    pltpu.sync_copy(beta_hbm, bbuf)
    bv = [bbuf[p, pl.ds(0, L)] for p in range(P)]
    start = _worker_start(wid)

    def rows(buf, zb):
        def row_body(r, c):
            for t in range(T):
                z = bv[0] * buf[r, pl.ds(t * L, L)]
                for p in range(1, P):
                    z = z + bv[p] * buf[r, pl.ds(p * D + t * L, L)]
                zb[r, pl.ds(t * L, L)] = z
            return c
        lax.fori_loop(0, CH, row_body, 0)

    def out_start(cid, zb, osem):
        pltpu.async_copy(zb, z_hbm.at[pl.ds(cid * CH, CH), :], osem)

    def out_wait(zb, osem):
        pltpu.make_async_copy(zb, z_hbm.at[pl.ds(0, CH), :], osem).wait()

    _in_start(h_hbm, start, buf0, sem0)

    def body2(k, c):
        g = 2 * k
        _in_start(h_hbm, start + g + 1, buf1, sem1)
        _in_wait(h_hbm, buf0, sem0)

        @pl.when(k > 0)
        def _():
            out_wait(zb0, osem0)
        rows(buf0, zb0)
        out_start(start + g, zb0, osem0)
        _in_start(h_hbm, start + g + 2, buf0, sem0)
        _in_wait(h_hbm, buf1, sem1)

        @pl.when(k > 0)
        def _():
            out_wait(zb1, osem1)
        rows(buf1, zb1)
        out_start(start + g + 1, zb1, osem1)
        return c

    lax.fori_loop(0, HALF, body2, 0)
    _in_wait(h_hbm, buf0, sem0)
    out_wait(zb0, osem0)
    rows(buf0, zb0)
    out_start(start + BASE_CH - 1, zb0, osem0)

    @pl.when(wid < EXTRA)
    def _extra():
        pltpu.sync_copy(h_hbm.at[pl.ds((start + BASE_CH) * CH, CH), :], buf1)
        out_wait(zb1, osem1)
        rows(buf1, zb1)
        out_start(start + BASE_CH, zb1, osem1)

    out_wait(zb0, osem0)
    out_wait(zb1, osem1)


def _pass2(h2, beta_b):
    f = pl.kernel(
        _pass2_body,
        out_type=jax.ShapeDtypeStruct((N, D), jnp.float32),
        mesh=_sc_mesh(),
        scratch_types=[
            pltpu.VMEM((P, 128), jnp.float32),
            pltpu.VMEM((CH, ROW), jnp.float32),
            pltpu.VMEM((CH, ROW), jnp.float32),
            pltpu.VMEM((CH, D), jnp.float32),
            pltpu.VMEM((CH, D), jnp.float32),
            pltpu.SemaphoreType.DMA,
            pltpu.SemaphoreType.DMA,
            pltpu.SemaphoreType.DMA,
            pltpu.SemaphoreType.DMA,
        ],
    )
    return f(h2, beta_b)


def kernel(h, W):
    h2 = h.reshape(N, ROW)
    Wf = W.reshape(D)
    wpart = _pass1(h2, Wf)
    beta_b = _beta(wpart)
    return _pass2(h2, beta_b)


# speedup vs baseline: 1.8147x; 1.4321x over previous
import functools
import jax
import jax.numpy as jnp
from jax import lax
from jax.experimental import pallas as pl
from jax.experimental.pallas import tpu as pltpu
from jax.experimental.pallas import tpu_sc as plsc

N, P, D = 10000, 8, 256
ROW = P * D
L = 16
T = D // L
NC, NS = 2, 16
NW = NC * NS
CH = 16
NCHUNK = N // CH
BASE_CH = NCHUNK // NW
EXTRA = NCHUNK - BASE_CH * NW
HALF = (BASE_CH - 1) // 2


def _worker_start(wid):
    return wid * BASE_CH + jnp.minimum(wid, EXTRA)


def _sc_mesh():
    return plsc.VectorSubcoreMesh(core_axis_name="c", subcore_axis_name="s")


def _in_start(h_hbm, cid, buf, sem):
    pltpu.async_copy(h_hbm.at[pl.ds(cid * CH, CH)], buf, sem)


def _in_wait(h_hbm, buf, sem):
    pltpu.make_async_copy(h_hbm.at[pl.ds(0, CH)], buf, sem).wait()


def _pass1_body(h_hbm, w_hbm, out_hbm, wbuf, buf0, buf1, stage, sem0, sem1):
    wid = lax.axis_index("s") * NC + lax.axis_index("c")
    pltpu.sync_copy(w_hbm, wbuf)
    wv = [wbuf[pl.ds(t * L, L)] for t in range(T)]
    start = _worker_start(wid)

    def rows(buf, acc):
        def row_body(r, acc):
            acc = list(acc)
            for p in range(P):
                a = acc[p]
                for t in range(T):
                    a = a + buf[r, p, pl.ds(t * L, L)] * wv[t]
                acc[p] = a
            return tuple(acc)
        return lax.fori_loop(0, CH, row_body, acc)

    _in_start(h_hbm, start, buf0, sem0)

    def body2(k, acc):
        g = 2 * k
        _in_start(h_hbm, start + g + 1, buf1, sem1)
        _in_wait(h_hbm, buf0, sem0)
        acc = rows(buf0, acc)
        _in_start(h_hbm, start + g + 2, buf0, sem0)
        _in_wait(h_hbm, buf1, sem1)
        return rows(buf1, acc)

    acc0 = tuple(jnp.zeros((L,), jnp.float32) for _ in range(P))
    acc = lax.fori_loop(0, HALF, body2, acc0)
    _in_wait(h_hbm, buf0, sem0)
    acc = rows(buf0, acc)
    for p in range(P):
        stage[p, :] = acc[p]

    @pl.when(wid < EXTRA)
    def _extra():
        pltpu.sync_copy(h_hbm.at[pl.ds((start + BASE_CH) * CH, CH)], buf1)
        acc_e = rows(buf1, acc0)
        for p in range(P):
            stage[p, :] = stage[p, :] + acc_e[p]

    pltpu.sync_copy(stage, out_hbm.at[wid])


def _pass1(h2, Wf):
    f = pl.kernel(
        _pass1_body,
        out_type=jax.ShapeDtypeStruct((NW, P, L), jnp.float32),
        mesh=_sc_mesh(),
        scratch_types=[
            pltpu.VMEM((D,), jnp.float32),
            pltpu.VMEM((CH, P, D), jnp.float32),
            pltpu.VMEM((CH, P, D), jnp.float32),
            pltpu.VMEM((P, L), jnp.float32),
            pltpu.SemaphoreType.DMA,
            pltpu.SemaphoreType.DMA,
        ],
    )
    return f(h2, Wf)


def _beta_body(wpart_ref, beta_ref):
    w = jnp.sum(wpart_ref[...], axis=(0, 2)).reshape(P, 1)
    w = jnp.where(w >= 0, w, 0.01 * w)
    m = jnp.max(w, axis=0, keepdims=True)
    e = jnp.exp(w - m)
    beta = e / jnp.sum(e, axis=0, keepdims=True)
    beta_ref[...] = jnp.broadcast_to(beta, (P, 128))


def _beta(wpart):
    return pl.pallas_call(
        _beta_body,
        out_shape=jax.ShapeDtypeStruct((P, 128), jnp.float32),
    )(wpart)


def _pass2_body(h_hbm, beta_hbm, z_hbm, bbuf, buf0, buf1, zb0, zb1,
                sem0, sem1, osem0, osem1):
    wid = lax.axis_index("s") * NC + lax.axis_index("c")
    pltpu.sync_copy(beta_hbm, bbuf)
    bv = [bbuf[p, pl.ds(0, L)] for p in range(P)]
    start = _worker_start(wid)

    def rows(buf, zb):
        def row_body(r, c):
            for t in range(T):
                z = bv[0] * buf[r, 0, pl.ds(t * L, L)]
                for p in range(1, P):
                    z = z + bv[p] * buf[r, p, pl.ds(t * L, L)]
                zb[r, pl.ds(t * L, L)] = z
            return c
        lax.fori_loop(0, CH, row_body, 0)

    def out_start(cid, zb, osem):
        pltpu.async_copy(zb, z_hbm.at[pl.ds(cid * CH, CH), :], osem)

    def out_wait(zb, osem):
        pltpu.make_async_copy(zb, z_hbm.at[pl.ds(0, CH), :], osem).wait()

    _in_start(h_hbm, start, buf0, sem0)

    def body2(k, c):
        g = 2 * k
        _in_start(h_hbm, start + g + 1, buf1, sem1)
        _in_wait(h_hbm, buf0, sem0)

        @pl.when(k > 0)
        def _():
            out_wait(zb0, osem0)
        rows(buf0, zb0)
        out_start(start + g, zb0, osem0)
        _in_start(h_hbm, start + g + 2, buf0, sem0)
        _in_wait(h_hbm, buf1, sem1)

        @pl.when(k > 0)
        def _():
            out_wait(zb1, osem1)
        rows(buf1, zb1)
        out_start(start + g + 1, zb1, osem1)
        return c

    lax.fori_loop(0, HALF, body2, 0)
    _in_wait(h_hbm, buf0, sem0)
    out_wait(zb0, osem0)
    rows(buf0, zb0)
    out_start(start + BASE_CH - 1, zb0, osem0)

    @pl.when(wid < EXTRA)
    def _extra():
        pltpu.sync_copy(h_hbm.at[pl.ds((start + BASE_CH) * CH, CH)], buf1)
        out_wait(zb1, osem1)
        rows(buf1, zb1)
        out_start(start + BASE_CH, zb1, osem1)

    out_wait(zb0, osem0)
    out_wait(zb1, osem1)


def _pass2(h2, beta_b):
    f = pl.kernel(
        _pass2_body,
        out_type=jax.ShapeDtypeStruct((N, D), jnp.float32),
        mesh=_sc_mesh(),
        scratch_types=[
            pltpu.VMEM((P, 128), jnp.float32),
            pltpu.VMEM((CH, P, D), jnp.float32),
            pltpu.VMEM((CH, P, D), jnp.float32),
            pltpu.VMEM((CH, D), jnp.float32),
            pltpu.VMEM((CH, D), jnp.float32),
            pltpu.SemaphoreType.DMA,
            pltpu.SemaphoreType.DMA,
            pltpu.SemaphoreType.DMA,
            pltpu.SemaphoreType.DMA,
        ],
    )
    return f(h2, beta_b)


def kernel(h, W):
    Wf = W.reshape(D)
    wpart = _pass1(h, Wf)
    beta_b = _beta(wpart)
    return _pass2(h, beta_b)
